# baseline (speedup 1.0000x reference)
import functools

import jax
import jax.numpy as jnp
from jax import lax
from jax.experimental import pallas as pl
from jax.experimental.pallas import tpu as pltpu
from jax.experimental.pallas import tpu_sc as plsc

_NUM_CORES = 2
_NUM_SUBCORES = 16
_NUM_WORKERS = _NUM_CORES * _NUM_SUBCORES
_LANES = 16


@functools.cache
def _make_reverse_kernel(M, N, R):
    rows_per_w = M // _NUM_WORKERS
    nblk = rows_per_w // R
    nchunks = N // _LANES
    mesh = plsc.VectorSubcoreMesh(core_axis_name="c", subcore_axis_name="s")

    @functools.partial(
        pl.kernel,
        out_type=jax.ShapeDtypeStruct((M, N), jnp.float32),
        mesh=mesh,
        scratch_types=[
            pltpu.VMEM((R, N), jnp.float32),
            pltpu.VMEM((R, N), jnp.float32),
        ],
    )
    def k(in_hbm, out_hbm, buf_in, buf_out):
        wid = lax.axis_index("s") * _NUM_CORES + lax.axis_index("c")
        base_row = wid * rows_per_w

        def blk_body(blk, carry):
            row0 = base_row + blk * R
            pltpu.sync_copy(in_hbm.at[pl.ds(row0, R)], buf_in)
            for r in range(R):
                def chunk_body(c, carry2):
                    v = buf_in[r, pl.ds(N - (c + 1) * _LANES, _LANES)]
                    buf_out[r, pl.ds(c * _LANES, _LANES)] = lax.rev(
                        v, dimensions=(0,))
                    return carry2
                lax.fori_loop(0, nchunks, chunk_body, 0, unroll=8)
            pltpu.sync_copy(buf_out, out_hbm.at[pl.ds(row0, R)])
            return carry

        lax.fori_loop(0, nblk, blk_body, 0)

    return k


def kernel(inputs):
    M, N = inputs.shape
    return _make_reverse_kernel(M, N, 4)(inputs)


# speedup vs baseline: 1.4091x; 1.4091x over previous
import functools

import jax
import jax.numpy as jnp
from jax import lax
from jax.experimental import pallas as pl
from jax.experimental.pallas import tpu as pltpu
from jax.experimental.pallas import tpu_sc as plsc

_NUM_CORES = 2
_NUM_SUBCORES = 16
_NUM_WORKERS = _NUM_CORES * _NUM_SUBCORES
_LANES = 16


@functools.cache
def _make_reverse_kernel(M, N, R):
    rows_per_w = M // _NUM_WORKERS
    nblk = rows_per_w // R
    nchunks = N // _LANES
    assert nblk % 2 == 0
    mesh = plsc.VectorSubcoreMesh(core_axis_name="c", subcore_axis_name="s")

    @functools.partial(
        pl.kernel,
        out_type=jax.ShapeDtypeStruct((M, N), jnp.float32),
        mesh=mesh,
        scratch_types=[
            pltpu.VMEM((R, N), jnp.float32),
            pltpu.VMEM((R, N), jnp.float32),
            pltpu.VMEM((R, N), jnp.float32),
            pltpu.VMEM((R, N), jnp.float32),
            pltpu.SemaphoreType.DMA,
            pltpu.SemaphoreType.DMA,
            pltpu.SemaphoreType.DMA,
            pltpu.SemaphoreType.DMA,
        ],
    )
    def k(in_hbm, out_hbm, bi0, bi1, bo0, bo1, si0, si1, so0, so1):
        wid = lax.axis_index("s") * _NUM_CORES + lax.axis_index("c")
        base_row = wid * rows_per_w

        def start_in(blk, buf, sem):
            pltpu.async_copy(in_hbm.at[pl.ds(base_row + blk * R, R)], buf, sem)

        def wait_in(buf, sem):
            pltpu.make_async_copy(in_hbm.at[pl.ds(base_row, R)], buf, sem).wait()

        def start_out(blk, buf, sem):
            pltpu.async_copy(buf, out_hbm.at[pl.ds(base_row + blk * R, R)], sem)

        def wait_out(buf, sem):
            pltpu.make_async_copy(buf, out_hbm.at[pl.ds(base_row, R)], sem).wait()

        def compute(buf_in, buf_out):
            for r in range(R):
                def chunk_body(c, carry):
                    v = buf_in[r, pl.ds(N - (c + 1) * _LANES, _LANES)]
                    buf_out[r, pl.ds(c * _LANES, _LANES)] = lax.rev(
                        v, dimensions=(0,))
                    return carry
                lax.fori_loop(0, nchunks, chunk_body, 0, unroll=8)

        start_in(0, bi0, si0)
        start_in(1, bi1, si1)

        def pair_body(p, carry):
            blk = 2 * p
            for par, (bi, bo, si, so) in enumerate(
                    ((bi0, bo0, si0, so0), (bi1, bo1, si1, so1))):
                b = blk + par
                wait_in(bi, si)

                @pl.when(p > 0)
                def _():
                    wait_out(bo, so)

                compute(bi, bo)
                start_out(b, bo, so)

                @pl.when(b + 2 < nblk)
                def _():
                    start_in(b + 2, bi, si)
            return carry

        lax.fori_loop(0, nblk // 2, pair_body, 0)
        wait_out(bo0, so0)
        wait_out(bo1, so1)

    return k


def kernel(inputs):
    M, N = inputs.shape
    return _make_reverse_kernel(M, N, 4)(inputs)


# speedup vs baseline: 4.0291x; 2.8593x over previous
import functools

import jax
import jax.numpy as jnp
from jax import lax
from jax.experimental import pallas as pl
from jax.experimental.pallas import tpu as pltpu
from jax.experimental.pallas import tpu_sc as plsc

_NUM_CORES = 2
_NUM_SUBCORES = 16
_NUM_WORKERS = _NUM_CORES * _NUM_SUBCORES
_LANES = 16


@functools.cache
def _make_reverse_kernel(M, N, R):
    rows_per_w = M // _NUM_WORKERS
    nblk = rows_per_w // R
    nchunks = N // _LANES
    assert nblk % 2 == 0
    mesh = plsc.VectorSubcoreMesh(core_axis_name="c", subcore_axis_name="s")

    @functools.partial(
        pl.kernel,
        out_type=jax.ShapeDtypeStruct((M, N), jnp.float32),
        mesh=mesh,
        scratch_types=[
            pltpu.VMEM((R, N), jnp.float32),
            pltpu.VMEM((R, N), jnp.float32),
            pltpu.VMEM((R, N), jnp.float32),
            pltpu.VMEM((R, N), jnp.float32),
            pltpu.SemaphoreType.DMA,
            pltpu.SemaphoreType.DMA,
            pltpu.SemaphoreType.DMA,
            pltpu.SemaphoreType.DMA,
        ],
    )
    def k(in_hbm, out_hbm, bi0, bi1, bo0, bo1, si0, si1, so0, so1):
        wid = lax.axis_index("s") * _NUM_CORES + lax.axis_index("c")
        base_row = wid * rows_per_w

        def start_in(blk, buf, sem):
            pltpu.async_copy(in_hbm.at[pl.ds(base_row + blk * R, R)], buf, sem)

        def wait_in(buf, sem):
            pltpu.make_async_copy(in_hbm.at[pl.ds(base_row, R)], buf, sem).wait()

        def start_out(blk, buf, sem):
            pltpu.async_copy(buf, out_hbm.at[pl.ds(base_row + blk * R, R)], sem)

        def wait_out(buf, sem):
            pltpu.make_async_copy(buf, out_hbm.at[pl.ds(base_row, R)], sem).wait()

        def compute(buf_in, buf_out):
            for r in range(R):
                @plsc.parallel_loop(0, N, step=_LANES, unroll=8)
                def _(c):
                    v = buf_in[r, pl.ds(N - _LANES - c, _LANES)]
                    buf_out[r, pl.ds(c, _LANES)] = lax.rev(v, dimensions=(0,))

        start_in(0, bi0, si0)
        start_in(1, bi1, si1)

        def pair_body(p, carry):
            blk = 2 * p
            for par, (bi, bo, si, so) in enumerate(
                    ((bi0, bo0, si0, so0), (bi1, bo1, si1, so1))):
                b = blk + par
                wait_in(bi, si)

                @pl.when(p > 0)
                def _():
                    wait_out(bo, so)

                compute(bi, bo)
                start_out(b, bo, so)

                @pl.when(b + 2 < nblk)
                def _():
                    start_in(b + 2, bi, si)
            return carry

        lax.fori_loop(0, nblk // 2, pair_body, 0)
        wait_out(bo0, so0)
        wait_out(bo1, so1)

    return k


def kernel(inputs):
    M, N = inputs.shape
    return _make_reverse_kernel(M, N, 4)(inputs)


# speedup vs baseline: 4.1225x; 1.0232x over previous
import functools

import jax
import jax.numpy as jnp
from jax import lax
from jax.experimental import pallas as pl
from jax.experimental.pallas import tpu as pltpu
from jax.experimental.pallas import tpu_sc as plsc

_NUM_CORES = 2
_NUM_SUBCORES = 16
_NUM_WORKERS = _NUM_CORES * _NUM_SUBCORES
_LANES = 16


@functools.cache
def _make_reverse_kernel(M, N, R, depth):
    rows_per_w = M // _NUM_WORKERS
    nblk = rows_per_w // R
    assert nblk % depth == 0
    mesh = plsc.VectorSubcoreMesh(core_axis_name="c", subcore_axis_name="s")

    scratch = ([pltpu.VMEM((R, N), jnp.float32)] * (2 * depth)
               + [pltpu.SemaphoreType.DMA] * (2 * depth))

    @functools.partial(
        pl.kernel,
        out_type=jax.ShapeDtypeStruct((M, N), jnp.float32),
        mesh=mesh,
        scratch_types=scratch,
    )
    def k(in_hbm, out_hbm, *refs):
        bi = refs[:depth]
        bo = refs[depth:2 * depth]
        si = refs[2 * depth:3 * depth]
        so = refs[3 * depth:4 * depth]

        wid = lax.axis_index("s") * _NUM_CORES + lax.axis_index("c")
        base_row = wid * rows_per_w

        def start_in(blk, d):
            pltpu.async_copy(
                in_hbm.at[pl.ds(base_row + blk * R, R)], bi[d], si[d])

        def wait_in(d):
            pltpu.make_async_copy(
                in_hbm.at[pl.ds(base_row, R)], bi[d], si[d]).wait()

        def start_out(blk, d):
            pltpu.async_copy(
                bo[d], out_hbm.at[pl.ds(base_row + blk * R, R)], so[d])

        def wait_out(d):
            pltpu.make_async_copy(
                bo[d], out_hbm.at[pl.ds(base_row, R)], so[d]).wait()

        def compute(buf_in, buf_out):
            for r in range(R):
                @plsc.parallel_loop(0, N, step=_LANES, unroll=8)
                def _(c):
                    v = buf_in[r, pl.ds(N - _LANES - c, _LANES)]
                    buf_out[r, pl.ds(c, _LANES)] = lax.rev(v, dimensions=(0,))

        for d in range(depth):
            start_in(d, d)

        def group_body(g, carry):
            blk0 = depth * g
            for d in range(depth):
                b = blk0 + d
                wait_in(d)

                @pl.when(g > 0)
                def _():
                    wait_out(d)

                compute(bi[d], bo[d])
                start_out(b, d)

                @pl.when(b + depth < nblk)
                def _():
                    start_in(b + depth, d)
            return carry

        lax.fori_loop(0, nblk // depth, group_body, 0)
        for d in range(depth):
            wait_out(d)

    return k


def kernel(inputs):
    M, N = inputs.shape
    return _make_reverse_kernel(M, N, 2, 4)(inputs)


# speedup vs baseline: 4.1589x; 1.0088x over previous
import functools

import jax
import jax.numpy as jnp
from jax import lax
from jax.experimental import pallas as pl
from jax.experimental.pallas import tpu as pltpu
from jax.experimental.pallas import tpu_sc as plsc

_NUM_CORES = 2
_NUM_SUBCORES = 16
_NUM_WORKERS = _NUM_CORES * _NUM_SUBCORES
_LANES = 16


@functools.cache
def _make_reverse_kernel(M, N, R, depth):
    rows_per_w = M // _NUM_WORKERS
    nblk = rows_per_w // R
    assert nblk % depth == 0
    mesh = plsc.VectorSubcoreMesh(core_axis_name="c", subcore_axis_name="s")

    scratch = ([pltpu.VMEM((R, N), jnp.float32)] * (2 * depth)
               + [pltpu.SemaphoreType.DMA] * (2 * depth))

    @functools.partial(
        pl.kernel,
        out_type=jax.ShapeDtypeStruct((M, N), jnp.float32),
        mesh=mesh,
        scratch_types=scratch,
    )
    def k(in_hbm, out_hbm, *refs):
        bi = refs[:depth]
        bo = refs[depth:2 * depth]
        si = refs[2 * depth:3 * depth]
        so = refs[3 * depth:4 * depth]

        wid = lax.axis_index("s") * _NUM_CORES + lax.axis_index("c")
        base_row = wid * rows_per_w

        def start_in(blk, d):
            pltpu.async_copy(
                in_hbm.at[pl.ds(base_row + blk * R, R)], bi[d], si[d])

        def wait_in(d):
            pltpu.make_async_copy(
                in_hbm.at[pl.ds(base_row, R)], bi[d], si[d]).wait()

        def start_out(blk, d):
            pltpu.async_copy(
                bo[d], out_hbm.at[pl.ds(base_row + blk * R, R)], so[d])

        def wait_out(d):
            pltpu.make_async_copy(
                bo[d], out_hbm.at[pl.ds(base_row, R)], so[d]).wait()

        def compute(buf_in, buf_out):
            for r in range(R):
                @plsc.parallel_loop(0, N, step=_LANES, unroll=8)
                def _(c):
                    v = buf_in[r, pl.ds(N - _LANES - c, _LANES)]
                    buf_out[r, pl.ds(c, _LANES)] = lax.rev(v, dimensions=(0,))

        for d in range(depth):
            start_in(d, d)

        def group_body(g, carry):
            blk0 = depth * g
            for d in range(depth):
                b = blk0 + d
                wait_in(d)

                @pl.when(g > 0)
                def _():
                    wait_out(d)

                compute(bi[d], bo[d])
                start_out(b, d)

                @pl.when(b + depth < nblk)
                def _():
                    start_in(b + depth, d)
            return carry

        lax.fori_loop(0, nblk // depth, group_body, 0)
        for d in range(depth):
            wait_out(d)

    return k


def kernel(inputs):
    M, N = inputs.shape
    return _make_reverse_kernel(M, N, 2, 8)(inputs)
